# baseline (speedup 1.0000x reference)
import functools

import numpy as np
import jax
import jax.numpy as jnp
from jax import lax
from jax.experimental import pallas as pl
from jax.experimental.pallas import tpu as pltpu
from jax.experimental.pallas import tpu_sc as plsc

_VOCAB = 1000000
_EMB = 64
_SEQ = 200
_BATCH = 4096
_NC = 2
_NS = 16
_NW = _NC * _NS
_BT = _BATCH // _NW
_EPS = 1e-3
_NB = 4


def _pos_table() -> np.ndarray:
    pos = np.arange(_SEQ)[:, np.newaxis]
    i = np.arange(_EMB)[np.newaxis, :]
    angle = pos * (1.0 / np.power(10000, 2 * (i // 2) / np.float32(_EMB)))
    angle[:, 0::2] = np.sin(angle[:, 0::2])
    angle[:, 1::2] = np.cos(angle[:, 1::2])
    return angle.astype(np.float32)


_POS_C = _pos_table().reshape(_SEQ, _EMB, 1)

_mesh = plsc.VectorSubcoreMesh(
    core_axis_name="c", subcore_axis_name="s", num_cores=_NC, num_subcores=_NS
)


@functools.partial(
    pl.kernel,
    out_type=jax.ShapeDtypeStruct((_SEQ, 2048, 128), jnp.float32),
    mesh=_mesh,
    compiler_params=pltpu.CompilerParams(
        needs_layout_passes=False, use_tc_tiling_on_sc=False
    ),
    scratch_types=[
        pltpu.VMEM((_SEQ, _BT), jnp.int32),
        pltpu.VMEM((2, _BT, _EMB), jnp.float32),
        pltpu.VMEM((2, _BT // 2, 128), jnp.float32),
        pltpu.SemaphoreType.DMA,
        pltpu.SemaphoreType.DMA,
    ],
)
def _sc_gather(idx_hbm, table_hbm, out_hbm, idx_v, inb, outb, gsem, ssem):
    wid = lax.axis_index("s") * _NC + lax.axis_index("c")

    pltpu.sync_copy(idx_hbm.at[:, wid], idx_v)

    def gather_start(g, b):
        pltpu.async_copy(table_hbm.at[idx_v.at[g]], inb.at[b], gsem)

    def gather_wait(g, b):
        pltpu.make_async_copy(
            table_hbm.at[idx_v.at[g]], inb.at[b], gsem
        ).wait()

    def store_start(g, b):
        pltpu.async_copy(
            outb.at[b], out_hbm.at[g, pl.ds(wid * (_BT // 2), _BT // 2)], ssem
        )

    def store_wait(g, b):
        pltpu.make_async_copy(
            outb.at[b], out_hbm.at[g, pl.ds(wid * (_BT // 2), _BT // 2)], ssem
        ).wait()

    def repack(b):
        ib = inb.at[b]
        ob = outb.at[b]

        @plsc.parallel_loop(0, _BT // 2, step=1, unroll=4)
        def pair(p):
            r0 = 2 * p
            for jg in range(4):
                ob[p, pl.ds(jg * 16, 16)] = ib[r0, pl.ds(jg * 16, 16)]
            for jg in range(4):
                ob[p, pl.ds(64 + jg * 16, 16)] = ib[r0 + 1, pl.ds(jg * 16, 16)]

    gather_start(0, 0)
    gather_start(1, 1)

    def step(i, carry):
        for b in (0, 1):
            g = i * 2 + b

            @pl.when(i > 0)
            def _():
                store_wait(g - 2, b)

            gather_wait(g, b)
            repack(b)
            store_start(g, b)

            @pl.when(g + 2 < _SEQ)
            def _():
                gather_start(g + 2, b)
        return carry

    lax.fori_loop(0, _SEQ // 2, step, 0)

    store_wait(_SEQ - 2, 0)
    store_wait(_SEQ - 1, 1)


_WC = 8


_DN_T = (((0,), (0,)), ((), ()))
_DN_M = (((1,), (0,)), ((), ()))


def _tc_ln_body(g_ref, pos_ref, gam_ref, bet_ref, o_ref):
    pos = pos_ref[0]
    gam = gam_ref[:, :]
    bet = bet_ref[:, :]
    eye = jnp.eye(_EMB, dtype=jnp.float32)
    ones = jnp.ones((1, _EMB), dtype=jnp.float32)
    f = jnp.float32
    for q in range(_WC):
        x = g_ref[0, pl.ds(q * 64, 64), :]
        t0 = lax.dot_general(x[:, :_EMB], eye, _DN_T,
                             preferred_element_type=jnp.float32)
        t1 = lax.dot_general(x[:, _EMB:], eye, _DN_T,
                             preferred_element_type=jnp.float32)
        xt = jnp.concatenate([t0, t1], axis=1)
        u = xt + pos
        s1 = lax.dot_general(ones, u, _DN_M,
                             preferred_element_type=jnp.float32)
        s2 = lax.dot_general(ones, u * u, _DN_M,
                             preferred_element_type=jnp.float32)
        m = s1 * f(1.0 / _EMB)
        var = s2 * f(1.0 / _EMB) - m * m
        r = lax.rsqrt(var + f(_EPS))
        y = (u - m) * r * gam + bet
        o_ref[0, :, q] = y.reshape(8, 8, 128)


_tc_ln = pl.pallas_call(
    _tc_ln_body,
    grid=(_SEQ, _NW // _WC),
    in_specs=[
        pl.BlockSpec((1, _WC * 64, 128), lambda s, w: (s, w, 0)),
        pl.BlockSpec((1, _EMB, 1), lambda s, w: (s, 0, 0)),
        pl.BlockSpec((_EMB, 1), lambda s, w: (0, 0)),
        pl.BlockSpec((_EMB, 1), lambda s, w: (0, 0)),
    ],
    out_specs=pl.BlockSpec((1, 8, _WC, 8, 128), lambda s, w: (s, 0, w, 0, 0)),
    out_shape=jax.ShapeDtypeStruct((_SEQ, 8, _NW, 8, 128), jnp.float32),
)


def kernel(input_ids, token_type_ids, table, gamma, beta):
    del token_type_ids
    idx = (
        input_ids.astype(jnp.int32)
        .T.reshape(_SEQ, _NW, 2, _BT // 2)
        .transpose(0, 1, 3, 2)
        .reshape(_SEQ, _NW, _BT)
    )
    g = _sc_gather(idx, table)
    out5 = _tc_ln(
        g,
        jnp.asarray(_POS_C),
        gamma.reshape(_EMB, 1),
        beta.reshape(_EMB, 1),
    )
    return out5.transpose(2, 4, 0, 1, 3).reshape(_BATCH, _SEQ, _EMB)


# speedup vs baseline: 1.6055x; 1.6055x over previous
import functools

import numpy as np
import jax
import jax.numpy as jnp
from jax import lax
from jax.experimental import pallas as pl
from jax.experimental.pallas import tpu as pltpu
from jax.experimental.pallas import tpu_sc as plsc

_VOCAB = 1000000
_EMB = 64
_SEQ = 200
_BATCH = 4096
_NC = 2
_NS = 16
_NW = _NC * _NS
_BT = _BATCH // _NW
_EPS = 1e-3
_L = 16


def _pos_centered() -> np.ndarray:
    pos = np.arange(_SEQ)[:, np.newaxis]
    i = np.arange(_EMB)[np.newaxis, :]
    angle = pos * (1.0 / np.power(10000, 2 * (i // 2) / np.float32(_EMB)))
    angle[:, 0::2] = np.sin(angle[:, 0::2])
    angle[:, 1::2] = np.cos(angle[:, 1::2])
    p = angle.astype(np.float32)
    return p - p.mean(axis=1, keepdims=True)


_POS = _pos_centered()

_mesh = plsc.VectorSubcoreMesh(
    core_axis_name="c", subcore_axis_name="s", num_cores=_NC, num_subcores=_NS
)


@functools.partial(
    pl.kernel,
    out_type=jax.ShapeDtypeStruct((_SEQ, 8, _NW, 8, _BT), jnp.float32),
    mesh=_mesh,
    compiler_params=pltpu.CompilerParams(
        needs_layout_passes=False, use_tc_tiling_on_sc=False
    ),
    scratch_types=[
        pltpu.VMEM((_SEQ // 8, 8, _BT), jnp.int32),
        pltpu.VMEM((_SEQ, _EMB), jnp.float32),
        pltpu.VMEM((_EMB,), jnp.float32),
        pltpu.VMEM((_EMB,), jnp.float32),
        pltpu.VMEM((2, _BT, _EMB), jnp.float32),
        pltpu.VMEM((2, 8, 8, _BT), jnp.float32),
        pltpu.VMEM((_BT, _EMB), jnp.float32),
        pltpu.VMEM((_BT, _L), jnp.float32),
        pltpu.VMEM((_BT, _L), jnp.float32),
        pltpu.VMEM((_BT,), jnp.float32),
        pltpu.VMEM((_BT,), jnp.float32),
        pltpu.SemaphoreType.DMA,
        pltpu.SemaphoreType.DMA,
    ],
)
def _emb_ln(idx_hbm, pos_hbm, gam_hbm, bet_hbm, table_hbm, out_hbm,
            idx_v, pos_v, gam_v, bet_v, inb, outb, ubuf, sbuf, qbuf,
            mbuf, rbuf, gsem, ssem):
    wid = lax.axis_index("s") * _NC + lax.axis_index("c")

    pltpu.sync_copy(idx_hbm.at[:, wid], idx_v)
    pltpu.sync_copy(pos_hbm, pos_v)
    pltpu.sync_copy(gam_hbm, gam_v)
    pltpu.sync_copy(bet_hbm, bet_v)

    def gather_start(g, b):
        ts = lax.div(g, 8)
        sr = lax.rem(g, 8)
        pltpu.async_copy(
            table_hbm.at[idx_v.at[ts, sr]], inb.at[b], gsem
        )

    def gather_wait(g, b):
        ts = lax.div(g, 8)
        sr = lax.rem(g, 8)
        pltpu.make_async_copy(
            table_hbm.at[idx_v.at[ts, sr]], inb.at[b], gsem
        ).wait()

    def store_start(g, b):
        pltpu.async_copy(
            outb.at[b], out_hbm.at[g, :, wid], ssem
        )

    def store_wait(g, b):
        pltpu.make_async_copy(
            outb.at[b], out_hbm.at[g, :, wid], ssem
        ).wait()

    gs = [gam_v[pl.ds(j * _L, _L)] for j in range(4)]
    bs = [bet_v[pl.ds(j * _L, _L)] for j in range(4)]

    lane = lax.iota(jnp.int32, _L)
    lane15 = jnp.full((_L,), 15, jnp.int32)
    tjv = [(lane + 16 * jg) >> 3 for jg in range(4)]
    jrv = [(lane + 16 * jg) & 7 for jg in range(4)]

    def compute(g, b):
        ib = inb.at[b]
        ob = outb.at[b]
        ps = [pos_v[g, pl.ds(jg * _L, _L)] for jg in range(4)]

        @plsc.parallel_loop(0, _BT, step=1, unroll=4)
        def row_a(r):
            u = []
            for jg in range(4):
                e = ib[r, pl.ds(jg * _L, _L)]
                u.append(e + ps[jg])
            for jg in range(4):
                ubuf[r, pl.ds(jg * _L, _L)] = u[jg]
            s = (u[0] + u[1]) + (u[2] + u[3])
            q = (u[0] * u[0] + u[1] * u[1]) + (u[2] * u[2] + u[3] * u[3])
            sbuf[r, pl.ds(0, _L)] = plsc.cumsum(s)
            qbuf[r, pl.ds(0, _L)] = plsc.cumsum(q)

        @plsc.parallel_loop(0, _BT // _L, step=1, unroll=2)
        def group(i):
            r0 = i * _L
            rows = r0 + lane
            ssum = plsc.load_gather(sbuf, [rows, lane15])
            qsum = plsc.load_gather(qbuf, [rows, lane15])
            mean = ssum * jnp.float32(1.0 / _EMB)
            x = qsum * jnp.float32(1.0 / _EMB) - mean * mean + jnp.float32(_EPS)
            ii = plsc.bitcast(x, jnp.int32)
            ii = jnp.int32(0x5F3759DF) - (ii >> 1)
            y = plsc.bitcast(ii, jnp.float32)
            hx = jnp.float32(0.5) * x
            for _ in range(2):
                y = y * (jnp.float32(1.5) - hx * y * y)
            mbuf[pl.ds(r0, _L)] = mean
            rbuf[pl.ds(r0, _L)] = y

        @plsc.parallel_loop(0, _BT, step=1, unroll=4)
        def row_c(r):
            rr = jnp.full((_L,), r, jnp.int32)
            mean = plsc.load_gather(mbuf, [rr])
            rstd = plsc.load_gather(rbuf, [rr])
            for jg in range(4):
                u = ubuf[r, pl.ds(jg * _L, _L)]
                y = (u - mean) * rstd * gs[jg] + bs[jg]
                plsc.store_scatter(ob, [tjv[jg], jrv[jg], rr], y)

    gather_start(0, 0)
    gather_start(1, 1)

    def step(i, carry):
        for b in (0, 1):
            g = i * 2 + b

            @pl.when(i > 0)
            def _():
                store_wait(g - 2, b)

            gather_wait(g, b)
            compute(g, b)
            store_start(g, b)

            @pl.when(g + 2 < _SEQ)
            def _():
                gather_start(g + 2, b)
        return carry

    lax.fori_loop(0, _SEQ // 2, step, 0)

    store_wait(_SEQ - 2, 0)
    store_wait(_SEQ - 1, 1)


def kernel(input_ids, token_type_ids, table, gamma, beta):
    del token_type_ids
    idx = (
        input_ids.T.reshape(_SEQ // 8, 8, _NW, _BT)
        .transpose(0, 2, 1, 3)
        .astype(jnp.int32)
    )
    out5 = _emb_ln(idx, jnp.asarray(_POS), gamma, beta, table)
    return out5.transpose(2, 4, 0, 1, 3).reshape(_BATCH, _SEQ, _EMB)
